# baseline (speedup 1.0000x reference)
import functools

import jax
import jax.numpy as jnp
from jax import lax
from jax.experimental import pallas as pl
from jax.experimental.pallas import tpu as pltpu
from jax.experimental.pallas import tpu_sc as plsc

_PLACEHOLDER_TOKEN = 42
_B, _N, _D = 1024, 77, 768
_NC, _NS, _L = 2, 16, 16
_NW = _NC * _NS
_ROWS_W = _B // _NW
_TOK_W = _ROWS_W * _N
_WIN = (0, 16, 32, 48, 61)


def _sc_body(tok_hbm, emb_hbm, ph_hbm, out_hbm, tok_v, ph_v,
             buf0, buf1, isems, osems):
    wid = lax.axis_index("s") * _NC + lax.axis_index("c")
    base_b = wid * _ROWS_W
    base_t = wid * _TOK_W

    pltpu.sync_copy(tok_hbm.at[pl.ds(base_t, _TOK_W)], tok_v)
    pltpu.sync_copy(ph_hbm, ph_v)

    lanes = lax.iota(jnp.int32, _L)
    zeros = lanes * 0
    bufs = (buf0, buf1)

    def _halves(r, s, sems):
        src = emb_hbm.at[pl.ds(base_b + r, 1)]
        return (
            pltpu.make_async_copy(src.at[:, pl.ds(0, 40)],
                                  bufs[s].at[:, pl.ds(0, 40)], sems.at[s, 0]),
            pltpu.make_async_copy(src.at[:, pl.ds(40, 37)],
                                  bufs[s].at[:, pl.ds(40, 37)], sems.at[s, 1]),
        )

    def _ohalves(r, s, sems):
        dst = out_hbm.at[pl.ds(base_b + r, 1)]
        return (
            pltpu.make_async_copy(bufs[s].at[:, pl.ds(0, 40)],
                                  dst.at[:, pl.ds(0, 40)], sems.at[s, 0]),
            pltpu.make_async_copy(bufs[s].at[:, pl.ds(40, 37)],
                                  dst.at[:, pl.ds(40, 37)], sems.at[s, 1]),
        )

    class _Pair:
        def __init__(self, copies):
            self._c = copies

        def start(self):
            for c in self._c:
                c.start()

        def wait(self):
            for c in self._c:
                c.wait()

    def in_copy(r, s):
        return _Pair(_halves(r, s, isems))

    def out_copy(r, s):
        return _Pair(_ohalves(r, s, osems))

    def fix_row(g, s):
        for start in _WIN:
            idx = lanes + (g * _N + start)
            v = plsc.load_gather(tok_v, [idx])
            mask = (v == _PLACEHOLDER_TOKEN).astype(jnp.int32)
            n16 = lanes + start

            def fix_cond(rem):
                return jnp.sum(rem) > 0

            def fix_body(rem):
                f = plsc.all_reduce_ffs(rem > 0)
                hit = lanes == f
                nsel = jnp.sum(jnp.where(hit, n16, 0))
                i1 = zeros + nsel
                for c in range(_D // _L):
                    chunk = ph_v[pl.ds(c * _L, _L)]
                    plsc.store_scatter(bufs[s], [zeros, i1, lanes + c * _L],
                                       chunk)
                return jnp.where(hit, 0, rem)

            lax.while_loop(fix_cond, fix_body, mask)

    in_copy(0, 0).start()

    def pair_body(g2, _):
        for s in (0, 1):
            r = g2 * 2 + s
            in_copy(r, s).wait()

            @pl.when(r >= 1)
            def _():
                out_copy(r - 1, 1 - s).wait()

            @pl.when(r + 1 < _ROWS_W)
            def _():
                in_copy(r + 1, 1 - s).start()

            fix_row(r, s)
            out_copy(r, s).start()
        return 0

    lax.fori_loop(0, _ROWS_W // 2, pair_body, jnp.int32(0))
    out_copy(_ROWS_W - 1, 1).wait()


def kernel(tokenized_text, embedded_text, placeholder_embedding):
    tok_flat = tokenized_text.reshape(_B * _N)
    ph1 = placeholder_embedding.reshape(_D)
    mesh = plsc.VectorSubcoreMesh(
        core_axis_name="c", subcore_axis_name="s",
        num_cores=_NC, num_subcores=_NS,
    )
    run = functools.partial(
        pl.kernel,
        out_type=jax.ShapeDtypeStruct((_B, _N, _D), embedded_text.dtype),
        mesh=mesh,
        compiler_params=pltpu.CompilerParams(needs_layout_passes=False),
        scratch_types=[
            pltpu.VMEM((_TOK_W,), jnp.int32),
            pltpu.VMEM((_D,), jnp.float32),
            pltpu.VMEM((1, _N, _D), jnp.float32),
            pltpu.VMEM((1, _N, _D), jnp.float32),
            pltpu.SemaphoreType.DMA((2, 2)),
            pltpu.SemaphoreType.DMA((2, 2)),
        ],
    )(_sc_body)
    return run(tok_flat, embedded_text, ph1)


# speedup vs baseline: 1.0005x; 1.0005x over previous
import functools

import jax
import jax.numpy as jnp
from jax import lax
from jax.experimental import pallas as pl
from jax.experimental.pallas import tpu as pltpu
from jax.experimental.pallas import tpu_sc as plsc

_PLACEHOLDER_TOKEN = 42
_B, _N, _D = 1024, 77, 768
_NC, _NS, _L = 2, 16, 16
_NW = _NC * _NS
_ROWS_W = _B // _NW
_TOK_W = _ROWS_W * _N
_WIN = (0, 16, 32, 48, 61)


def _sc_body(tok_hbm, emb_hbm, ph_hbm, out_hbm, tok_v, ph_v,
             buf0, buf1, isems, osems):
    wid = lax.axis_index("s") * _NC + lax.axis_index("c")
    base_b = wid * _ROWS_W
    base_t = wid * _TOK_W

    pltpu.sync_copy(tok_hbm.at[pl.ds(base_t, _TOK_W)], tok_v)
    pltpu.sync_copy(ph_hbm, ph_v)

    lanes = lax.iota(jnp.int32, _L)
    zeros = lanes * 0
    bufs = (buf0, buf1)

    def in_copy(r, s):
        return pltpu.make_async_copy(emb_hbm.at[pl.ds(base_b + r, 1)],
                                bufs[s], isems.at[s])

    def out_copy(r, s):
        return pltpu.make_async_copy(bufs[s], out_hbm.at[pl.ds(base_b + r, 1)],
                                osems.at[s])

    def fix_row(g, s):
        for start in _WIN:
            idx = lanes + (g * _N + start)
            v = plsc.load_gather(tok_v, [idx])
            mask = (v == _PLACEHOLDER_TOKEN).astype(jnp.int32)
            n16 = lanes + start

            def fix_cond(rem):
                return jnp.sum(rem) > 0

            def fix_body(rem):
                f = plsc.all_reduce_ffs(rem > 0)
                hit = lanes == f
                nsel = jnp.sum(jnp.where(hit, n16, 0))
                i1 = zeros + nsel
                for c in range(_D // _L):
                    chunk = ph_v[pl.ds(c * _L, _L)]
                    plsc.store_scatter(bufs[s], [zeros, i1, lanes + c * _L],
                                       chunk)
                return jnp.where(hit, 0, rem)

            lax.while_loop(fix_cond, fix_body, mask)

    in_copy(0, 0).start()

    def pair_body(g2, _):
        for s in (0, 1):
            r = g2 * 2 + s
            in_copy(r, s).wait()

            @pl.when(r >= 1)
            def _():
                out_copy(r - 1, 1 - s).wait()

            @pl.when(r + 1 < _ROWS_W)
            def _():
                in_copy(r + 1, 1 - s).start()

            fix_row(r, s)
            out_copy(r, s).start()
        return 0

    lax.fori_loop(0, _ROWS_W // 2, pair_body, jnp.int32(0))
    out_copy(_ROWS_W - 1, 1).wait()


def kernel(tokenized_text, embedded_text, placeholder_embedding):
    tok_flat = tokenized_text.reshape(_B * _N)
    ph1 = placeholder_embedding.reshape(_D)
    mesh = plsc.VectorSubcoreMesh(
        core_axis_name="c", subcore_axis_name="s",
        num_cores=_NC, num_subcores=_NS,
    )
    run = functools.partial(
        pl.kernel,
        out_type=jax.ShapeDtypeStruct((_B, _N, _D), embedded_text.dtype),
        mesh=mesh,
        compiler_params=pltpu.CompilerParams(needs_layout_passes=False),
        scratch_types=[
            pltpu.VMEM((_TOK_W,), jnp.int32),
            pltpu.VMEM((_D,), jnp.float32),
            pltpu.VMEM((1, _N, _D), jnp.float32),
            pltpu.VMEM((1, _N, _D), jnp.float32),
            pltpu.SemaphoreType.DMA((2,)),
            pltpu.SemaphoreType.DMA((2,)),
        ],
    )(_sc_body)
    return run(tok_flat, embedded_text, ph1)
